# baseline (speedup 1.0000x reference)
import jax
import jax.numpy as jnp
from jax import lax
from jax.experimental import pallas as pl
from jax.experimental.pallas import tpu as pltpu
from jax.experimental.pallas import tpu_sc as plsc

DS = 16
D = 112
D_MIN = 2.0
D_INV_INT = 2.0
POS_MAX = 112.0 - 1e-06
SENTINEL = 100000.0

B, V, H, W = 8, 6, 256, 704
BV = B * V
HP = H // DS
WP = W // DS
NCOLS = BV * WP
NWORKERS = 32
COLS_PER_W = NCOLS // NWORKERS
CHUNK = 22
NCHUNKS = COLS_PER_W // CHUNK
CHUNK_ROWS = CHUNK * HP
LANES = 128


def _min_body(sel_ref, x_ref, o_ref):
    x = x_ref[...].reshape(2 * V * W, H)
    t = jnp.where(x == 0.0, SENTINEL, x)
    rows = []
    for g in range(2 * V * WP):
        blk = t[g * DS:(g + 1) * DS]
        m8 = jnp.minimum(blk[0:8], blk[8:16])
        m4 = jnp.minimum(m8[0:4], m8[4:8])
        m2 = jnp.minimum(m4[0:2], m4[2:4])
        rows.append(jnp.minimum(m2[0:1], m2[1:2]))
    r = jnp.concatenate(rows, axis=0)
    for k in (1, 2, 4, 8):
        pad = jnp.full((2 * V * WP, k), SENTINEL, jnp.float32)
        r = jnp.minimum(r, jnp.concatenate([r[:, k:], pad], axis=1))
    sel = sel_ref[...]
    hi = r.astype(jnp.bfloat16)
    rem = r - hi.astype(jnp.float32)
    mid = rem.astype(jnp.bfloat16)
    lo = (rem - mid.astype(jnp.float32)).astype(jnp.bfloat16)
    dn = (((1,), (0,)), ((), ()))
    acc = lax.dot_general(hi, sel, dn, preferred_element_type=jnp.float32)
    acc = acc + lax.dot_general(mid, sel, dn,
                                preferred_element_type=jnp.float32)
    acc = acc + lax.dot_general(lo, sel, dn,
                                preferred_element_type=jnp.float32)
    o_ref[...] = acc


def _sc_body(minmap, out, inmin, outbuf):
    cid = lax.axis_index("c")
    sid = lax.axis_index("s")
    wid = sid * 2 + cid

    iota = lax.iota(jnp.int32, 16)
    zeros16 = jnp.zeros((16,), jnp.float32)

    pltpu.sync_copy(minmap.at[pl.ds(wid * COLS_PER_W, COLS_PER_W)], inmin)

    def zrow(r, c2):
        for j in range(D // 16):
            outbuf[r, pl.ds(j * 16, 16)] = zeros16
        return c2

    lax.fori_loop(0, CHUNK_ROWS, zrow, 0)

    def chunk_body(k, carry):
        sites = []
        for col in range(CHUNK):
            c_local = k * CHUNK + col
            m = inmin[c_local, pl.ds(0, 16)]
            pos = jnp.clip((m - D_MIN) * D_INV_INT, 0.0, POS_MAX)
            lower = pos.astype(jnp.int32)
            upper = jnp.minimum(lower + 1, D - 1)
            w_upper = jnp.clip(pos - lower.astype(jnp.float32), 0.0, 1.0)
            validf = jnp.where(m < SENTINEL, 1.0, 0.0)
            w_lower = (1.0 - w_upper) * validf
            w_upper = w_upper * validf

            rows = col * HP + iota
            plsc.addupdate_scatter(outbuf, [rows, lower], w_lower)
            plsc.addupdate_scatter(outbuf, [rows, upper], w_upper)
            sites.append((rows, lower, upper))

        base = (wid * COLS_PER_W + k * CHUNK) * HP
        pltpu.sync_copy(
            outbuf.at[:, pl.ds(0, D)],
            out.at[pl.ds(base, CHUNK_ROWS), pl.ds(0, D)])

        for rows, lower, upper in sites:
            plsc.store_scatter(outbuf, [rows, lower], zeros16)
            plsc.store_scatter(outbuf, [rows, upper], zeros16)
        return carry

    lax.fori_loop(0, NCHUNKS, chunk_body, 0)


@jax.jit
def kernel(lidar_depth):
    xt = jnp.transpose(lidar_depth, (0, 1, 3, 2))
    ci = lax.broadcasted_iota(jnp.int32, (H, LANES), 0)
    ji = lax.broadcasted_iota(jnp.int32, (H, LANES), 1)
    sel = jnp.where((ci == ji * DS) & (ji < HP), 1.0, 0.0).astype(jnp.bfloat16)
    minmap = pl.pallas_call(
        _min_body,
        grid=(B // 2,),
        in_specs=[
            pl.BlockSpec((H, LANES), lambda i: (0, 0)),
            pl.BlockSpec((2, V, W, H), lambda i: (i, 0, 0, 0)),
        ],
        out_specs=pl.BlockSpec((2 * V * WP, LANES), lambda i: (i, 0)),
        out_shape=jax.ShapeDtypeStruct((NCOLS, LANES), jnp.float32),
    )(sel, xt)

    mesh = plsc.VectorSubcoreMesh(core_axis_name="c", subcore_axis_name="s")
    f = pl.kernel(
        _sc_body,
        out_type=jax.ShapeDtypeStruct((NCOLS * HP, LANES), jnp.float32),
        mesh=mesh,
        scratch_types=[
            pltpu.VMEM((COLS_PER_W, LANES), jnp.float32),
            pltpu.VMEM((CHUNK_ROWS, LANES), jnp.float32),
        ],
        compiler_params=pltpu.CompilerParams(
            use_tc_tiling_on_sc=False, needs_layout_passes=False
        ),
    )
    y = f(minmap)
    y = y.reshape(BV, WP, HP, LANES)[..., :D]
    return jnp.transpose(y, (0, 3, 2, 1))


# speedup vs baseline: 1.0075x; 1.0075x over previous
import jax
import jax.numpy as jnp
from jax import lax
from jax.experimental import pallas as pl
from jax.experimental.pallas import tpu as pltpu
from jax.experimental.pallas import tpu_sc as plsc

DS = 16
D = 112
D_MIN = 2.0
D_INV_INT = 2.0
POS_MAX = 112.0 - 1e-06
SENTINEL = 100000.0

B, V, H, W = 8, 6, 256, 704
BV = B * V
HP = H // DS
WP = W // DS
NCOLS = BV * WP
NWORKERS = 32
COLS_PER_W = NCOLS // NWORKERS
CHUNK = 22
NCHUNKS = COLS_PER_W // CHUNK
CHUNK_ROWS = CHUNK * HP
LANES = 128


def _min_body(sel_ref, x_ref, o_ref):
    x = x_ref[...].reshape(2 * V * W, H)
    t = jnp.where(x == 0.0, SENTINEL, x)
    rows = []
    for g in range(2 * V * WP):
        blk = t[g * DS:(g + 1) * DS]
        m8 = jnp.minimum(blk[0:8], blk[8:16])
        m4 = jnp.minimum(m8[0:4], m8[4:8])
        m2 = jnp.minimum(m4[0:2], m4[2:4])
        rows.append(jnp.minimum(m2[0:1], m2[1:2]))
    r = jnp.concatenate(rows, axis=0)
    for k in (1, 2, 4, 8):
        pad = jnp.full((2 * V * WP, k), SENTINEL, jnp.float32)
        r = jnp.minimum(r, jnp.concatenate([r[:, k:], pad], axis=1))
    sel = sel_ref[...]
    hi = r.astype(jnp.bfloat16)
    rem = r - hi.astype(jnp.float32)
    mid = rem.astype(jnp.bfloat16)
    lo = (rem - mid.astype(jnp.float32)).astype(jnp.bfloat16)
    dn = (((1,), (0,)), ((), ()))
    acc = lax.dot_general(hi, sel, dn, preferred_element_type=jnp.float32)
    acc = acc + lax.dot_general(mid, sel, dn,
                                preferred_element_type=jnp.float32)
    acc = acc + lax.dot_general(lo, sel, dn,
                                preferred_element_type=jnp.float32)
    o_ref[...] = acc


def _sc_body(minmap, out, inmin, outbuf):
    cid = lax.axis_index("c")
    sid = lax.axis_index("s")
    wid = sid * 2 + cid

    iota = lax.iota(jnp.int32, 16)
    zeros16 = jnp.zeros((16,), jnp.float32)

    pltpu.sync_copy(minmap.at[pl.ds(wid * COLS_PER_W, COLS_PER_W)], inmin)

    def zrow(r, c2):
        for j in range(D // 16):
            outbuf[r, pl.ds(j * 16, 16)] = zeros16
        return c2

    lax.fori_loop(0, CHUNK_ROWS, zrow, 0)

    def chunk_body(k, carry):
        sites = []
        for col in range(CHUNK):
            c_local = k * CHUNK + col
            m = inmin[c_local, pl.ds(0, 16)]
            pos = jnp.clip((m - D_MIN) * D_INV_INT, 0.0, POS_MAX)
            lower = pos.astype(jnp.int32)
            upper = jnp.minimum(lower + 1, D - 1)
            w_upper = jnp.clip(pos - lower.astype(jnp.float32), 0.0, 1.0)
            validf = jnp.where(m < SENTINEL, 1.0, 0.0)
            w_lower = (1.0 - w_upper) * validf
            w_upper = w_upper * validf

            rows = col * HP + iota
            plsc.addupdate_scatter(outbuf, [rows, lower], w_lower)
            plsc.addupdate_scatter(outbuf, [rows, upper], w_upper)
            sites.append((rows, lower, upper))

        base = (wid * COLS_PER_W + k * CHUNK) * HP
        pltpu.sync_copy(outbuf, out.at[pl.ds(base, CHUNK_ROWS)])

        for rows, lower, upper in sites:
            plsc.store_scatter(outbuf, [rows, lower], zeros16)
            plsc.store_scatter(outbuf, [rows, upper], zeros16)
        return carry

    lax.fori_loop(0, NCHUNKS, chunk_body, 0)


@jax.jit
def kernel(lidar_depth):
    xt = jnp.transpose(lidar_depth, (0, 1, 3, 2))
    ci = lax.broadcasted_iota(jnp.int32, (H, LANES), 0)
    ji = lax.broadcasted_iota(jnp.int32, (H, LANES), 1)
    sel = jnp.where((ci == ji * DS) & (ji < HP), 1.0, 0.0).astype(jnp.bfloat16)
    minmap = pl.pallas_call(
        _min_body,
        grid=(B // 2,),
        in_specs=[
            pl.BlockSpec((H, LANES), lambda i: (0, 0)),
            pl.BlockSpec((2, V, W, H), lambda i: (i, 0, 0, 0)),
        ],
        out_specs=pl.BlockSpec((2 * V * WP, LANES), lambda i: (i, 0)),
        out_shape=jax.ShapeDtypeStruct((NCOLS, LANES), jnp.float32),
    )(sel, xt)

    mesh = plsc.VectorSubcoreMesh(core_axis_name="c", subcore_axis_name="s")
    f = pl.kernel(
        _sc_body,
        out_type=jax.ShapeDtypeStruct((NCOLS * HP, LANES), jnp.float32),
        mesh=mesh,
        scratch_types=[
            pltpu.VMEM((COLS_PER_W, LANES), jnp.float32),
            pltpu.VMEM((CHUNK_ROWS, LANES), jnp.float32),
        ],
        compiler_params=pltpu.CompilerParams(
            use_tc_tiling_on_sc=False, needs_layout_passes=False
        ),
    )
    y = f(minmap)
    y = y.reshape(BV, WP, HP, LANES)[..., :D]
    return jnp.transpose(y, (0, 3, 2, 1))
